# baseline (speedup 1.0000x reference)
import functools

import jax
import jax.numpy as jnp
from jax import lax
from jax.experimental import pallas as pl
from jax.experimental.pallas import tpu as pltpu
from jax.experimental.pallas import tpu_sc as plsc

NC = 2
NS = 16
NW = NC * NS
L = 16

V = 1000000
D = 16
LINES = V // 8
GRP = 1024
NFULL = V // GRP
NPAIR = (NFULL // NW + 2) // 2

_PARAMS = pltpu.CompilerParams(
    use_tc_tiling_on_sc=True, needs_layout_passes=False
)


def _transpose_panel(pan, out, nrows):
    lane = lax.iota(jnp.int32, L)

    def rowblk(j, c):
        for m in range(8):
            r = j * 8 + m
            row = plsc.load_gather(pan, [lane, jnp.full((L,), 0, jnp.int32) + r])
            out[j, pl.ds(m * L, L)] = row
        return c

    lax.fori_loop(0, nrows // 8, rowblk, 0)


def _k1_body(tabT_hbm, tail_hbm, tabL_hbm,
             pan0, pan1, out0, out1, sin0, sin1, sout0, sout1):
    wid = lax.axis_index("s") * NC + lax.axis_index("c")
    ngrp = jnp.where(wid < NFULL % NW, NFULL // NW + 1, NFULL // NW)
    pans = (pan0, pan1)
    outs = (out0, out1)
    sins = (sin0, sin1)
    souts = (sout0, sout1)

    def start_in(i, pan, sem):
        r0 = pl.multiple_of((wid + i * NW) * GRP, 128)
        pltpu.async_copy(tabT_hbm.at[:, pl.ds(r0, GRP)], pan.at[:, pl.ds(0, GRP)], sem)

    start_in(0, pans[0], sins[0])

    def pair(j, c):
        for p in (0, 1):
            i = j * 2 + p

            @pl.when(i < ngrp)
            def _():
                pltpu.make_async_copy(
                    tabT_hbm.at[:, pl.ds(0, GRP)],
                    pans[p].at[:, pl.ds(0, GRP)],
                    sins[p],
                ).wait()

                @pl.when(i + 1 < ngrp)
                def _():
                    start_in(i + 1, pans[1 - p], sins[1 - p])

                @pl.when(i >= 2)
                def _():
                    pltpu.make_async_copy(
                        outs[p], tabL_hbm.at[pl.ds(0, GRP // 8)], souts[p]
                    ).wait()

                _transpose_panel(pans[p], outs[p], GRP)
                q0 = pl.multiple_of((wid + i * NW) * (GRP // 8), 8)
                pltpu.async_copy(
                    outs[p], tabL_hbm.at[pl.ds(q0, GRP // 8)], souts[p]
                )
        return c

    lax.fori_loop(0, NPAIR, pair, 0)
    for p in (0, 1):
        pltpu.make_async_copy(
            outs[p], tabL_hbm.at[pl.ds(0, GRP // 8)], souts[p]
        ).wait()

    @pl.when(wid == 0)
    def _():
        r0 = pl.multiple_of(NFULL * GRP, 128)
        pltpu.sync_copy(tabT_hbm.at[:, pl.ds(r0, 512)], pan0.at[:, pl.ds(0, 512)])
        _transpose_panel(pan0, out0, 512)
        pltpu.sync_copy(
            out0.at[pl.ds(0, 64)],
            tabL_hbm.at[pl.ds(pl.multiple_of(NFULL * (GRP // 8), 8), 64)],
        )

    @pl.when(wid == 1)
    def _():
        pltpu.sync_copy(tail_hbm, out1.at[pl.ds(0, 8)])
        pltpu.sync_copy(out1.at[pl.ds(0, 8)], tabL_hbm.at[pl.ds(LINES - 8, 8)])


def _k2_body(f_cnt, bpw, tabL_hbm, idxT_hbm, out_hbm,
             idxblk, q0, q1, m0, m1, lines0, lines1, cols0, cols1,
             sg0, sg1, so0, so1):
    wid = lax.axis_index("s") * NC + lax.axis_index("c")
    b0 = pl.multiple_of(wid * bpw, 128)
    half = bpw // 2
    lane = lax.iota(jnp.int32, L)
    qs = (q0, q1)
    ms = (m0, m1)
    lines = (lines0, lines1)
    colss = (cols0, cols1)
    sgs = (sg0, sg1)
    sos = (so0, so1)

    pltpu.sync_copy(idxT_hbm.at[:, pl.ds(b0, bpw)], idxblk)

    def qcomp(f, q_v, m_v):
        def g_body(g, c):
            iv = idxblk[f, pl.ds(g * L, L)]
            q_v[pl.ds(g * L, L)] = iv >> 3
            m_v[pl.ds(g * L, L)] = iv & 7
            return c

        lax.fori_loop(0, bpw // L, g_body, 0)

    def extract(buf, m_v, co, cols):
        def e_body(i, c):
            mvec = m_v[pl.ds(co + i * L, L)]
            for k in range(L):
                row = buf[i * L + k, pl.ds(mvec[k] * L, L)]
                plsc.store_scatter(
                    cols,
                    [lane, jnp.full((L,), 0, jnp.int32) + (co + i * L + k)],
                    row,
                )
            return c

        lax.fori_loop(0, half // L, e_body, 0)

    def norms(cols):
        def n_body(g, c):
            bb = g * L
            colv = [cols[cc, pl.ds(bb, L)] for cc in range(D)]
            s = colv[0] * colv[0]
            for cc in range(1, D):
                s = s + colv[cc] * colv[cc]
            y = lax.bitcast_convert_type(
                jnp.full((L,), 0x5F3759DF, jnp.int32)
                - (lax.bitcast_convert_type(s, jnp.int32) >> 1),
                jnp.float32,
            )
            for _ in range(3):
                y = y * (1.5 - 0.5 * s * y * y)
            scale = jnp.where(s > 1.0, y, jnp.float32(1.0))
            for cc in range(D):
                cols[cc, pl.ds(bb, L)] = colv[cc] * scale
            return c

        lax.fori_loop(0, bpw // L, n_body, 0)

    def gather_half(q_v, co, buf, sem):
        pltpu.async_copy(tabL_hbm.at[q_v.at[pl.ds(co, half)]], buf, sem)

    def wait_g(buf, sem):
        pltpu.make_async_copy(tabL_hbm.at[pl.ds(0, half)], buf, sem).wait()

    qcomp(0, qs[0], ms[0])
    gather_half(qs[0], 0, lines[0], sgs[0])

    def fbody(p, f):
        a, b = lines[p], lines[1 - p]
        gather_half(qs[p], half, b, sgs[1 - p])
        wait_g(a, sgs[p])
        extract(a, ms[p], 0, colss[p])
        fn = jnp.minimum(f + 1, f_cnt - 1)
        qcomp(fn, qs[1 - p], ms[1 - p])
        wait_g(b, sgs[1 - p])
        extract(b, ms[p], half, colss[p])

        @pl.when(f + 1 < f_cnt)
        def _():
            gather_half(qs[1 - p], 0, b, sgs[1 - p])

        norms(colss[p])

        @pl.when(f >= 1)
        def _():
            pltpu.make_async_copy(
                colss[1 - p].at[:, pl.ds(0, bpw)],
                out_hbm.at[0, :, pl.ds(b0, bpw)],
                sos[1 - p],
            ).wait()

        pltpu.async_copy(
            colss[p].at[:, pl.ds(0, bpw)],
            out_hbm.at[f, :, pl.ds(b0, bpw)],
            sos[p],
        )

    def pair(j, c):
        fbody(0, j * 2)
        fbody(1, j * 2 + 1)
        return c

    lax.fori_loop(0, f_cnt // 2, pair, 0)
    pltpu.make_async_copy(
        colss[1].at[:, pl.ds(0, bpw)], out_hbm.at[0, :, pl.ds(b0, bpw)], sos[1]
    ).wait()


def kernel(idx, table):
    b, f_cnt = idx.shape
    v, d = table.shape
    assert (v, d) == (V, D) and b % NW == 0 and f_cnt % 2 == 0
    bpw = b // NW

    mesh = plsc.VectorSubcoreMesh(
        core_axis_name="c", subcore_axis_name="s", num_cores=NC, num_subcores=NS
    )

    k1 = functools.partial(
        pl.kernel,
        out_type=jax.ShapeDtypeStruct((LINES, 128), jnp.float32),
        mesh=mesh,
        scratch_types=[
            pltpu.VMEM((L, GRP + 1), jnp.float32),
            pltpu.VMEM((L, GRP + 1), jnp.float32),
            pltpu.VMEM((GRP // 8, 128), jnp.float32),
            pltpu.VMEM((GRP // 8, 128), jnp.float32),
            pltpu.SemaphoreType.DMA,
            pltpu.SemaphoreType.DMA,
            pltpu.SemaphoreType.DMA,
            pltpu.SemaphoreType.DMA,
        ],
        compiler_params=_PARAMS,
    )(_k1_body)

    k2 = functools.partial(
        pl.kernel,
        out_type=jax.ShapeDtypeStruct((f_cnt, D, b), jnp.float32),
        mesh=mesh,
        scratch_types=[
            pltpu.VMEM((f_cnt, bpw), jnp.int32),
            pltpu.VMEM((bpw,), jnp.int32),
            pltpu.VMEM((bpw,), jnp.int32),
            pltpu.VMEM((bpw,), jnp.int32),
            pltpu.VMEM((bpw,), jnp.int32),
            pltpu.VMEM((bpw // 2, 128), jnp.float32),
            pltpu.VMEM((bpw // 2, 128), jnp.float32),
            pltpu.VMEM((D, bpw + 1), jnp.float32),
            pltpu.VMEM((D, bpw + 1), jnp.float32),
            pltpu.SemaphoreType.DMA,
            pltpu.SemaphoreType.DMA,
            pltpu.SemaphoreType.DMA,
            pltpu.SemaphoreType.DMA,
        ],
        compiler_params=_PARAMS,
    )(functools.partial(_k2_body, f_cnt, bpw))

    tail = table[NFULL * GRP + 512 :, :].reshape(8, 128)
    tabL = k1(table.T, tail)
    out3 = k2(tabL, idx.T)
    return jnp.transpose(out3, (2, 0, 1))


# speedup vs baseline: 1.0008x; 1.0008x over previous
import functools

import jax
import jax.numpy as jnp
from jax import lax
from jax.experimental import pallas as pl
from jax.experimental.pallas import tpu as pltpu
from jax.experimental.pallas import tpu_sc as plsc

NC = 2
NS = 16
NW = NC * NS
L = 16

V = 1000000
D = 16
LINES = V // 8
GRP = 1024
NFULL = V // GRP
NPAIR = (NFULL // NW + 2) // 2

_PARAMS = pltpu.CompilerParams(
    use_tc_tiling_on_sc=True, needs_layout_passes=False
)


def _transpose_panel(pan, out, nrows):
    lane = lax.iota(jnp.int32, L)

    def rowblk(j, c):
        for m in range(8):
            r = j * 8 + m
            row = plsc.load_gather(pan, [lane, jnp.full((L,), 0, jnp.int32) + r])
            out[j, pl.ds(m * L, L)] = row
        return c

    lax.fori_loop(0, nrows // 8, rowblk, 0)


def _k1_body(tabT_hbm, tail_hbm, tabL_hbm,
             pan0, pan1, out0, out1, sin0, sin1, sout0, sout1):
    wid = lax.axis_index("s") * NC + lax.axis_index("c")
    ngrp = jnp.where(wid < NFULL % NW, NFULL // NW + 1, NFULL // NW)
    pans = (pan0, pan1)
    outs = (out0, out1)
    sins = (sin0, sin1)
    souts = (sout0, sout1)

    def start_in(i, pan, sem):
        r0 = pl.multiple_of((wid + i * NW) * GRP, 128)
        pltpu.async_copy(tabT_hbm.at[:, pl.ds(r0, GRP)], pan.at[:, pl.ds(0, GRP)], sem)

    start_in(0, pans[0], sins[0])

    def pair(j, c):
        for p in (0, 1):
            i = j * 2 + p

            @pl.when(i < ngrp)
            def _():
                pltpu.make_async_copy(
                    tabT_hbm.at[:, pl.ds(0, GRP)],
                    pans[p].at[:, pl.ds(0, GRP)],
                    sins[p],
                ).wait()

                @pl.when(i + 1 < ngrp)
                def _():
                    start_in(i + 1, pans[1 - p], sins[1 - p])

                @pl.when(i >= 2)
                def _():
                    pltpu.make_async_copy(
                        outs[p], tabL_hbm.at[pl.ds(0, GRP // 8)], souts[p]
                    ).wait()

                _transpose_panel(pans[p], outs[p], GRP)
                q0 = pl.multiple_of((wid + i * NW) * (GRP // 8), 8)
                pltpu.async_copy(
                    outs[p], tabL_hbm.at[pl.ds(q0, GRP // 8)], souts[p]
                )
        return c

    lax.fori_loop(0, NPAIR, pair, 0)
    for p in (0, 1):
        pltpu.make_async_copy(
            outs[p], tabL_hbm.at[pl.ds(0, GRP // 8)], souts[p]
        ).wait()

    @pl.when(wid == 0)
    def _():
        r0 = pl.multiple_of(NFULL * GRP, 128)
        pltpu.sync_copy(tabT_hbm.at[:, pl.ds(r0, 512)], pan0.at[:, pl.ds(0, 512)])
        _transpose_panel(pan0, out0, 512)
        pltpu.sync_copy(
            out0.at[pl.ds(0, 64)],
            tabL_hbm.at[pl.ds(pl.multiple_of(NFULL * (GRP // 8), 8), 64)],
        )

    @pl.when(wid == 1)
    def _():
        pltpu.sync_copy(tail_hbm, out1.at[pl.ds(0, 8)])
        pltpu.sync_copy(out1.at[pl.ds(0, 8)], tabL_hbm.at[pl.ds(LINES - 8, 8)])


def _k2_body(f_cnt, bpw, tabL_hbm, idxT_hbm, out_hbm,
             idxblk, q0, q1, m0, m1, lines0, lines1, cols0, cols1,
             sg0, sg1, so0, so1):
    wid = lax.axis_index("s") * NC + lax.axis_index("c")
    b0 = pl.multiple_of(wid * bpw, 128)
    half = bpw // 2
    lane = lax.iota(jnp.int32, L)
    qs = (q0, q1)
    ms = (m0, m1)
    lines = (lines0, lines1)
    colss = (cols0, cols1)
    sgs = (sg0, sg1)
    sos = (so0, so1)

    pltpu.sync_copy(idxT_hbm.at[:, pl.ds(b0, bpw)], idxblk)

    def qcomp(f, q_v, m_v):
        def g_body(g, c):
            iv = idxblk[f, pl.ds(g * L, L)]
            q_v[pl.ds(g * L, L)] = iv >> 3
            m_v[pl.ds(g * L, L)] = iv & 7
            return c

        lax.fori_loop(0, bpw // L, g_body, 0)

    def extract(buf, m_v, co, cols):
        def e_body(i, c):
            mvec = m_v[pl.ds(co + i * L, L)]
            for k in range(L):
                row = buf[i * L + k, pl.ds(mvec[k] * L, L)]
                plsc.store_scatter(
                    cols,
                    [lane, jnp.full((L,), 0, jnp.int32) + (co + i * L + k)],
                    row,
                )
            return c

        lax.fori_loop(0, half // L, e_body, 0)

    def norms(cols):
        def n_body(g, c):
            bb = g * L
            colv = [cols[cc, pl.ds(bb, L)] for cc in range(D)]
            s = colv[0] * colv[0]
            for cc in range(1, D):
                s = s + colv[cc] * colv[cc]
            y = lax.bitcast_convert_type(
                jnp.full((L,), 0x5F3759DF, jnp.int32)
                - (lax.bitcast_convert_type(s, jnp.int32) >> 1),
                jnp.float32,
            )
            for _ in range(3):
                y = y * (1.5 - 0.5 * s * y * y)
            scale = jnp.where(s > 1.0, y, jnp.float32(1.0))
            for cc in range(D):
                cols[cc, pl.ds(bb, L)] = colv[cc] * scale
            return c

        lax.fori_loop(0, bpw // L, n_body, 0)

    def gather_half(q_v, co, buf, sem):
        pltpu.async_copy(tabL_hbm.at[q_v.at[pl.ds(co, half)]], buf, sem)

    def wait_g(buf, sem):
        pltpu.make_async_copy(tabL_hbm.at[pl.ds(0, half)], buf, sem).wait()

    qcomp(0, qs[0], ms[0])
    gather_half(qs[0], 0, lines[0], sgs[0])

    def fbody(p, f):
        a, b = lines[p], lines[1 - p]
        gather_half(qs[p], half, b, sgs[1 - p])
        wait_g(a, sgs[p])
        extract(a, ms[p], 0, colss[p])
        fn = jnp.minimum(f + 1, f_cnt - 1)
        qcomp(fn, qs[1 - p], ms[1 - p])
        wait_g(b, sgs[1 - p])
        extract(b, ms[p], half, colss[p])

        @pl.when(f + 1 < f_cnt)
        def _():
            gather_half(qs[1 - p], 0, b, sgs[1 - p])

        norms(colss[p])

        @pl.when(f >= 1)
        def _():
            pltpu.make_async_copy(
                colss[1 - p].at[:, pl.ds(0, bpw)],
                out_hbm.at[0, :, pl.ds(b0, bpw)],
                sos[1 - p],
            ).wait()

        pltpu.async_copy(
            colss[p].at[:, pl.ds(0, bpw)],
            out_hbm.at[f, :, pl.ds(b0, bpw)],
            sos[p],
        )

    def pair(j, c):
        fbody(0, j * 2)
        fbody(1, j * 2 + 1)
        return c

    lax.fori_loop(0, f_cnt // 2, pair, 0)
    pltpu.make_async_copy(
        colss[1].at[:, pl.ds(0, bpw)], out_hbm.at[0, :, pl.ds(b0, bpw)], sos[1]
    ).wait()


def kernel(idx, table):
    b, f_cnt = idx.shape
    v, d = table.shape
    assert (v, d) == (V, D) and b % NW == 0 and f_cnt % 2 == 0
    bpw = b // NW

    mesh = plsc.VectorSubcoreMesh(
        core_axis_name="c", subcore_axis_name="s", num_cores=NC, num_subcores=NS
    )

    k1 = functools.partial(
        pl.kernel,
        out_type=jax.ShapeDtypeStruct((LINES, 128), jnp.float32),
        mesh=mesh,
        scratch_types=[
            pltpu.VMEM((L, GRP + 16), jnp.float32),
            pltpu.VMEM((L, GRP + 16), jnp.float32),
            pltpu.VMEM((GRP // 8, 128), jnp.float32),
            pltpu.VMEM((GRP // 8, 128), jnp.float32),
            pltpu.SemaphoreType.DMA,
            pltpu.SemaphoreType.DMA,
            pltpu.SemaphoreType.DMA,
            pltpu.SemaphoreType.DMA,
        ],
        compiler_params=_PARAMS,
    )(_k1_body)

    k2 = functools.partial(
        pl.kernel,
        out_type=jax.ShapeDtypeStruct((f_cnt, D, b), jnp.float32),
        mesh=mesh,
        scratch_types=[
            pltpu.VMEM((f_cnt, bpw), jnp.int32),
            pltpu.VMEM((bpw,), jnp.int32),
            pltpu.VMEM((bpw,), jnp.int32),
            pltpu.VMEM((bpw,), jnp.int32),
            pltpu.VMEM((bpw,), jnp.int32),
            pltpu.VMEM((bpw // 2, 128), jnp.float32),
            pltpu.VMEM((bpw // 2, 128), jnp.float32),
            pltpu.VMEM((D, bpw + 16), jnp.float32),
            pltpu.VMEM((D, bpw + 16), jnp.float32),
            pltpu.SemaphoreType.DMA,
            pltpu.SemaphoreType.DMA,
            pltpu.SemaphoreType.DMA,
            pltpu.SemaphoreType.DMA,
        ],
        compiler_params=_PARAMS,
    )(functools.partial(_k2_body, f_cnt, bpw))

    tail = table[NFULL * GRP + 512 :, :].reshape(8, 128)
    tabL = k1(table.T, tail)
    out3 = k2(tabL, idx.T)
    return jnp.transpose(out3, (2, 0, 1))


# speedup vs baseline: 2.2809x; 2.2790x over previous
import functools

import jax
import jax.numpy as jnp
from jax import lax
from jax.experimental import pallas as pl
from jax.experimental.pallas import tpu as pltpu
from jax.experimental.pallas import tpu_sc as plsc

NC = 2
NS = 16
NW = NC * NS
L = 16

V = 1000000
D = 16
LINES = V // 8
GRP = 1024
NFULL = V // GRP
NPAIR = (NFULL // NW + 2) // 2

_PARAMS = pltpu.CompilerParams(
    use_tc_tiling_on_sc=True, needs_layout_passes=False
)


def _butterfly16(vecs):
    lane = lax.iota(jnp.int32, L)
    for s in range(4):
        bit = 1 << s
        m = (lane & bit) == 0
        xi = lane ^ bit
        nv = list(vecs)
        for i in range(L):
            if i & bit == 0:
                t, u = vecs[i], vecs[i | bit]
                pt = t.at[xi].get(mode="promise_in_bounds")
                pu = u.at[xi].get(mode="promise_in_bounds")
                nv[i] = jnp.where(m, t, pu)
                nv[i | bit] = jnp.where(m, pt, u)
        vecs = nv
    return vecs


def _transpose_panel(pan, out, nrows):
    def rowblk(j, c):
        r0 = j * L
        vecs = [pan[cc, pl.ds(r0, L)] for cc in range(L)]
        tv = _butterfly16(vecs)
        for i in range(L):
            out[j * 2 + (i >> 3), pl.ds((i & 7) * L, L)] = tv[i]
        return c

    lax.fori_loop(0, nrows // L, rowblk, 0)


def _k1_body(tabT_hbm, tail_hbm, tabL_hbm,
             pan0, pan1, out0, out1, sin0, sin1, sout0, sout1):
    wid = lax.axis_index("s") * NC + lax.axis_index("c")
    ngrp = jnp.where(wid < NFULL % NW, NFULL // NW + 1, NFULL // NW)
    pans = (pan0, pan1)
    outs = (out0, out1)
    sins = (sin0, sin1)
    souts = (sout0, sout1)

    def start_in(i, pan, sem):
        r0 = pl.multiple_of((wid + i * NW) * GRP, 128)
        pltpu.async_copy(tabT_hbm.at[:, pl.ds(r0, GRP)], pan.at[:, pl.ds(0, GRP)], sem)

    start_in(0, pans[0], sins[0])

    def pair(j, c):
        for p in (0, 1):
            i = j * 2 + p

            @pl.when(i < ngrp)
            def _():
                pltpu.make_async_copy(
                    tabT_hbm.at[:, pl.ds(0, GRP)],
                    pans[p].at[:, pl.ds(0, GRP)],
                    sins[p],
                ).wait()

                @pl.when(i + 1 < ngrp)
                def _():
                    start_in(i + 1, pans[1 - p], sins[1 - p])

                @pl.when(i >= 2)
                def _():
                    pltpu.make_async_copy(
                        outs[p], tabL_hbm.at[pl.ds(0, GRP // 8)], souts[p]
                    ).wait()

                _transpose_panel(pans[p], outs[p], GRP)
                q0 = pl.multiple_of((wid + i * NW) * (GRP // 8), 8)
                pltpu.async_copy(
                    outs[p], tabL_hbm.at[pl.ds(q0, GRP // 8)], souts[p]
                )
        return c

    lax.fori_loop(0, NPAIR, pair, 0)
    for p in (0, 1):
        pltpu.make_async_copy(
            outs[p], tabL_hbm.at[pl.ds(0, GRP // 8)], souts[p]
        ).wait()

    @pl.when(wid == 0)
    def _():
        r0 = pl.multiple_of(NFULL * GRP, 128)
        pltpu.sync_copy(tabT_hbm.at[:, pl.ds(r0, 512)], pan0.at[:, pl.ds(0, 512)])
        _transpose_panel(pan0, out0, 512)
        pltpu.sync_copy(
            out0.at[pl.ds(0, 64)],
            tabL_hbm.at[pl.ds(pl.multiple_of(NFULL * (GRP // 8), 8), 64)],
        )

    @pl.when(wid == 1)
    def _():
        pltpu.sync_copy(tail_hbm, out1.at[pl.ds(0, 8)])
        pltpu.sync_copy(out1.at[pl.ds(0, 8)], tabL_hbm.at[pl.ds(LINES - 8, 8)])


def _k2_body(f_cnt, bpw, tabL_hbm, idxT_hbm, out_hbm,
             idxblk, q0, q1, m0, m1, lines0, lines1, cols0, cols1,
             sg0, sg1, so0, so1):
    wid = lax.axis_index("s") * NC + lax.axis_index("c")
    b0 = pl.multiple_of(wid * bpw, 128)
    half = bpw // 2
    lane = lax.iota(jnp.int32, L)
    qs = (q0, q1)
    ms = (m0, m1)
    lines = (lines0, lines1)
    colss = (cols0, cols1)
    sgs = (sg0, sg1)
    sos = (so0, so1)

    pltpu.sync_copy(idxT_hbm.at[:, pl.ds(b0, bpw)], idxblk)

    def qcomp(f, q_v, m_v):
        def g_body(g, c):
            iv = idxblk[f, pl.ds(g * L, L)]
            q_v[pl.ds(g * L, L)] = iv >> 3
            m_v[pl.ds(g * L, L)] = iv & 7
            return c

        lax.fori_loop(0, bpw // L, g_body, 0)

    def extract_norm(buf, m_v, co, cols):
        def e_body(i, c):
            mvec = m_v[pl.ds(co + i * L, L)]
            rows = [buf[i * L + k, pl.ds(mvec[k] * L, L)] for k in range(L)]
            tv = _butterfly16(rows)
            s = tv[0] * tv[0]
            for cc in range(1, D):
                s = s + tv[cc] * tv[cc]
            y = lax.bitcast_convert_type(
                jnp.full((L,), 0x5F3759DF, jnp.int32)
                - (lax.bitcast_convert_type(s, jnp.int32) >> 1),
                jnp.float32,
            )
            for _ in range(3):
                y = y * (1.5 - 0.5 * s * y * y)
            scale = jnp.where(s > 1.0, y, jnp.float32(1.0))
            for cc in range(D):
                cols[cc, pl.ds(co + i * L, L)] = tv[cc] * scale
            return c

        lax.fori_loop(0, half // L, e_body, 0)

    def gather_half(q_v, co, buf, sem):
        pltpu.async_copy(tabL_hbm.at[q_v.at[pl.ds(co, half)]], buf, sem)

    def wait_g(buf, sem):
        pltpu.make_async_copy(tabL_hbm.at[pl.ds(0, half)], buf, sem).wait()

    qcomp(0, qs[0], ms[0])
    gather_half(qs[0], 0, lines[0], sgs[0])

    def fbody(p, f):
        a, b = lines[p], lines[1 - p]
        gather_half(qs[p], half, b, sgs[1 - p])
        wait_g(a, sgs[p])
        extract_norm(a, ms[p], 0, colss[p])
        fn = jnp.minimum(f + 1, f_cnt - 1)
        qcomp(fn, qs[1 - p], ms[1 - p])
        wait_g(b, sgs[1 - p])
        extract_norm(b, ms[p], half, colss[p])

        @pl.when(f + 1 < f_cnt)
        def _():
            gather_half(qs[1 - p], 0, b, sgs[1 - p])


        @pl.when(f >= 1)
        def _():
            pltpu.make_async_copy(
                colss[1 - p].at[:, pl.ds(0, bpw)],
                out_hbm.at[0, :, pl.ds(b0, bpw)],
                sos[1 - p],
            ).wait()

        pltpu.async_copy(
            colss[p].at[:, pl.ds(0, bpw)],
            out_hbm.at[f, :, pl.ds(b0, bpw)],
            sos[p],
        )

    def pair(j, c):
        fbody(0, j * 2)
        fbody(1, j * 2 + 1)
        return c

    lax.fori_loop(0, f_cnt // 2, pair, 0)
    pltpu.make_async_copy(
        colss[1].at[:, pl.ds(0, bpw)], out_hbm.at[0, :, pl.ds(b0, bpw)], sos[1]
    ).wait()


def kernel(idx, table):
    b, f_cnt = idx.shape
    v, d = table.shape
    assert (v, d) == (V, D) and b % NW == 0 and f_cnt % 2 == 0
    bpw = b // NW

    mesh = plsc.VectorSubcoreMesh(
        core_axis_name="c", subcore_axis_name="s", num_cores=NC, num_subcores=NS
    )

    k1 = functools.partial(
        pl.kernel,
        out_type=jax.ShapeDtypeStruct((LINES, 128), jnp.float32),
        mesh=mesh,
        scratch_types=[
            pltpu.VMEM((L, GRP + 16), jnp.float32),
            pltpu.VMEM((L, GRP + 16), jnp.float32),
            pltpu.VMEM((GRP // 8, 128), jnp.float32),
            pltpu.VMEM((GRP // 8, 128), jnp.float32),
            pltpu.SemaphoreType.DMA,
            pltpu.SemaphoreType.DMA,
            pltpu.SemaphoreType.DMA,
            pltpu.SemaphoreType.DMA,
        ],
        compiler_params=_PARAMS,
    )(_k1_body)

    k2 = functools.partial(
        pl.kernel,
        out_type=jax.ShapeDtypeStruct((f_cnt, D, b), jnp.float32),
        mesh=mesh,
        scratch_types=[
            pltpu.VMEM((f_cnt, bpw), jnp.int32),
            pltpu.VMEM((bpw,), jnp.int32),
            pltpu.VMEM((bpw,), jnp.int32),
            pltpu.VMEM((bpw,), jnp.int32),
            pltpu.VMEM((bpw,), jnp.int32),
            pltpu.VMEM((bpw // 2, 128), jnp.float32),
            pltpu.VMEM((bpw // 2, 128), jnp.float32),
            pltpu.VMEM((D, bpw + 16), jnp.float32),
            pltpu.VMEM((D, bpw + 16), jnp.float32),
            pltpu.SemaphoreType.DMA,
            pltpu.SemaphoreType.DMA,
            pltpu.SemaphoreType.DMA,
            pltpu.SemaphoreType.DMA,
        ],
        compiler_params=_PARAMS,
    )(functools.partial(_k2_body, f_cnt, bpw))

    tail = table[NFULL * GRP + 512 :, :].reshape(8, 128)
    tabL = k1(table.T, tail)
    out3 = k2(tabL, idx.T)
    return jnp.transpose(out3, (2, 0, 1))


# speedup vs baseline: 3.0619x; 1.3424x over previous
import functools

import jax
import jax.numpy as jnp
from jax import lax
from jax.experimental import pallas as pl
from jax.experimental.pallas import tpu as pltpu
from jax.experimental.pallas import tpu_sc as plsc

NC = 2
NS = 16
NW = NC * NS
L = 16

V = 1000000
D = 16
LINES = V // 8
GRP = 1024
NFULL = V // GRP
NPAIR = (NFULL // NW + 2) // 2

_PARAMS = pltpu.CompilerParams(
    use_tc_tiling_on_sc=True, needs_layout_passes=False
)


def _butterfly16(vecs):
    lane = lax.iota(jnp.int32, L)
    for s in range(4):
        bit = 1 << s
        m = (lane & bit) == 0
        xi = lane ^ bit
        nv = list(vecs)
        for i in range(L):
            if i & bit == 0:
                t, u = vecs[i], vecs[i | bit]
                pt = t.at[xi].get(mode="promise_in_bounds")
                pu = u.at[xi].get(mode="promise_in_bounds")
                nv[i] = jnp.where(m, t, pu)
                nv[i | bit] = jnp.where(m, pt, u)
        vecs = nv
    return vecs


def _transpose_panel(pan, out, nrows):
    def rowblk(j, c):
        r0 = j * L
        vecs = [pan[cc, pl.ds(r0, L)] for cc in range(L)]
        tv = _butterfly16(vecs)
        for i in range(L):
            out[j * 2 + (i >> 3), pl.ds((i & 7) * L, L)] = tv[i]
        return c

    lax.fori_loop(0, nrows // L, rowblk, 0)


def _k1_body(tabT_hbm, tail_hbm, tabL_hbm,
             pan0, pan1, out0, out1, sin0, sin1, sout0, sout1):
    wid = lax.axis_index("s") * NC + lax.axis_index("c")
    ngrp = jnp.where(wid < NFULL % NW, NFULL // NW + 1, NFULL // NW)
    pans = (pan0, pan1)
    outs = (out0, out1)
    sins = (sin0, sin1)
    souts = (sout0, sout1)

    def start_in(i, pan, sem):
        r0 = pl.multiple_of((wid + i * NW) * GRP, 128)
        pltpu.async_copy(tabT_hbm.at[:, pl.ds(r0, GRP)], pan.at[:, pl.ds(0, GRP)], sem)

    start_in(0, pans[0], sins[0])

    def pair(j, c):
        for p in (0, 1):
            i = j * 2 + p

            @pl.when(i < ngrp)
            def _():
                pltpu.make_async_copy(
                    tabT_hbm.at[:, pl.ds(0, GRP)],
                    pans[p].at[:, pl.ds(0, GRP)],
                    sins[p],
                ).wait()

                @pl.when(i + 1 < ngrp)
                def _():
                    start_in(i + 1, pans[1 - p], sins[1 - p])

                @pl.when(i >= 2)
                def _():
                    pltpu.make_async_copy(
                        outs[p], tabL_hbm.at[pl.ds(0, GRP // 8)], souts[p]
                    ).wait()

                _transpose_panel(pans[p], outs[p], GRP)
                q0 = pl.multiple_of((wid + i * NW) * (GRP // 8), 8)
                pltpu.async_copy(
                    outs[p], tabL_hbm.at[pl.ds(q0, GRP // 8)], souts[p]
                )
        return c

    lax.fori_loop(0, NPAIR, pair, 0)
    for p in (0, 1):
        pltpu.make_async_copy(
            outs[p], tabL_hbm.at[pl.ds(0, GRP // 8)], souts[p]
        ).wait()

    @pl.when(wid == 0)
    def _():
        r0 = pl.multiple_of(NFULL * GRP, 128)
        pltpu.sync_copy(tabT_hbm.at[:, pl.ds(r0, 512)], pan0.at[:, pl.ds(0, 512)])
        _transpose_panel(pan0, out0, 512)
        pltpu.sync_copy(
            out0.at[pl.ds(0, 64)],
            tabL_hbm.at[pl.ds(pl.multiple_of(NFULL * (GRP // 8), 8), 64)],
        )

    @pl.when(wid == 1)
    def _():
        pltpu.sync_copy(tail_hbm, out1.at[pl.ds(0, 8)])
        pltpu.sync_copy(out1.at[pl.ds(0, 8)], tabL_hbm.at[pl.ds(LINES - 8, 8)])


def _k2_body(f_cnt, bpw, tabL_hbm, idxT_hbm, out_hbm,
             idxblk, q0, q1, m0, m1, lines0, lines1, cols0, cols1,
             sg0, sg1, so0, so1):
    wid = lax.axis_index("s") * NC + lax.axis_index("c")
    b0 = pl.multiple_of(wid * bpw, 128)
    half = bpw // 2
    lane = lax.iota(jnp.int32, L)
    qs = (q0, q1)
    ms = (m0, m1)
    lines = (lines0, lines1)
    colss = (cols0, cols1)
    sgs = (sg0, sg1)
    sos = (so0, so1)

    pltpu.sync_copy(idxT_hbm.at[:, pl.ds(b0, bpw)], idxblk)

    def qcomp(f, q_v, m_v):
        def g_body(g, c):
            iv = idxblk[f, pl.ds(g * L, L)]
            q_v[pl.ds(g * L, L)] = iv >> 3
            m_v[pl.ds(g * L, L)] = iv & 7
            return c

        lax.fori_loop(0, bpw // L, g_body, 0)

    def extract_norm(buf, m_v, co, cols):
        def e_body(i, c):
            mvec = m_v[pl.ds(co + i * L, L)]
            rows = [buf[i * L + k, pl.ds(mvec[k] * L, L)] for k in range(L)]
            tv = _butterfly16(rows)
            s = tv[0] * tv[0]
            for cc in range(1, D):
                s = s + tv[cc] * tv[cc]
            y = lax.bitcast_convert_type(
                jnp.full((L,), 0x5F3759DF, jnp.int32)
                - (lax.bitcast_convert_type(s, jnp.int32) >> 1),
                jnp.float32,
            )
            for _ in range(3):
                y = y * (1.5 - 0.5 * s * y * y)
            scale = jnp.where(s > 1.0, y, jnp.float32(1.0))
            for cc in range(D):
                cols[cc, pl.ds(co + i * L, L)] = tv[cc] * scale
            return c

        lax.fori_loop(0, half // L, e_body, 0)

    def gather_half(q_v, co, buf, sem):
        pltpu.async_copy(tabL_hbm.at[q_v.at[pl.ds(co, half)]], buf, sem)

    def wait_g(buf, sem):
        pltpu.make_async_copy(tabL_hbm.at[pl.ds(0, half)], buf, sem).wait()

    qcomp(0, qs[0], ms[0])
    gather_half(qs[0], 0, lines[0], sgs[0])

    def fbody(p, f):
        gather_half(qs[p], half, lines[1], sgs[1])
        wait_g(lines[0], sgs[0])
        extract_norm(lines[0], ms[p], 0, colss[p])
        fn = jnp.minimum(f + 1, f_cnt - 1)
        qcomp(fn, qs[1 - p], ms[1 - p])

        @pl.when(f + 1 < f_cnt)
        def _():
            gather_half(qs[1 - p], 0, lines[0], sgs[0])

        wait_g(lines[1], sgs[1])
        extract_norm(lines[1], ms[p], half, colss[p])


        @pl.when(f >= 1)
        def _():
            pltpu.make_async_copy(
                colss[1 - p].at[:, pl.ds(0, bpw)],
                out_hbm.at[0, :, pl.ds(b0, bpw)],
                sos[1 - p],
            ).wait()

        pltpu.async_copy(
            colss[p].at[:, pl.ds(0, bpw)],
            out_hbm.at[f, :, pl.ds(b0, bpw)],
            sos[p],
        )

    def pair(j, c):
        fbody(0, j * 2)
        fbody(1, j * 2 + 1)
        return c

    lax.fori_loop(0, f_cnt // 2, pair, 0)
    pltpu.make_async_copy(
        colss[1].at[:, pl.ds(0, bpw)], out_hbm.at[0, :, pl.ds(b0, bpw)], sos[1]
    ).wait()


def kernel(idx, table):
    b, f_cnt = idx.shape
    v, d = table.shape
    assert (v, d) == (V, D) and b % NW == 0 and f_cnt % 2 == 0
    bpw = b // NW

    mesh = plsc.VectorSubcoreMesh(
        core_axis_name="c", subcore_axis_name="s", num_cores=NC, num_subcores=NS
    )

    k1 = functools.partial(
        pl.kernel,
        out_type=jax.ShapeDtypeStruct((LINES, 128), jnp.float32),
        mesh=mesh,
        scratch_types=[
            pltpu.VMEM((L, GRP + 16), jnp.float32),
            pltpu.VMEM((L, GRP + 16), jnp.float32),
            pltpu.VMEM((GRP // 8, 128), jnp.float32),
            pltpu.VMEM((GRP // 8, 128), jnp.float32),
            pltpu.SemaphoreType.DMA,
            pltpu.SemaphoreType.DMA,
            pltpu.SemaphoreType.DMA,
            pltpu.SemaphoreType.DMA,
        ],
        compiler_params=_PARAMS,
    )(_k1_body)

    k2 = functools.partial(
        pl.kernel,
        out_type=jax.ShapeDtypeStruct((f_cnt, D, b), jnp.float32),
        mesh=mesh,
        scratch_types=[
            pltpu.VMEM((f_cnt, bpw), jnp.int32),
            pltpu.VMEM((bpw,), jnp.int32),
            pltpu.VMEM((bpw,), jnp.int32),
            pltpu.VMEM((bpw,), jnp.int32),
            pltpu.VMEM((bpw,), jnp.int32),
            pltpu.VMEM((bpw // 2, 128), jnp.float32),
            pltpu.VMEM((bpw // 2, 128), jnp.float32),
            pltpu.VMEM((D, bpw + 16), jnp.float32),
            pltpu.VMEM((D, bpw + 16), jnp.float32),
            pltpu.SemaphoreType.DMA,
            pltpu.SemaphoreType.DMA,
            pltpu.SemaphoreType.DMA,
            pltpu.SemaphoreType.DMA,
        ],
        compiler_params=_PARAMS,
    )(functools.partial(_k2_body, f_cnt, bpw))

    tail = table[NFULL * GRP + 512 :, :].reshape(8, 128)
    tabL = k1(table.T, tail)
    out3 = k2(tabL, idx.T)
    return jnp.transpose(out3, (2, 0, 1))
